# baseline (speedup 1.0000x reference)
import functools

import jax
import jax.numpy as jnp
from jax import lax
from jax.experimental import pallas as pl
from jax.experimental.pallas import tpu as pltpu
from jax.experimental.pallas import tpu_sc as plsc

NUM_FIELDS = 26
VOCAB = 100000
EMB = 16
BATCH = 16384
INPUT_DIM = NUM_FIELDS * EMB

NC, NS, L = 2, 16, 16
NW = NC * NS
TOTAL = BATCH * NUM_FIELDS
PER_W = TOTAL // NW
CHUNK = 128
NCHUNK = PER_W // CHUNK


def _sc_gather_body(x_hbm, t1_hbm, t2_hbm, out2_hbm, out1_hbm,
                    x_v, idx_v, rows_v, e1_v, sem_x, sem2, sem1):
    wid = lax.axis_index("s") * NC + lax.axis_index("c")
    base = wid * PER_W

    pltpu.sync_copy(x_hbm.at[pl.ds(base, PER_W)], x_v)
    iota = lax.iota(jnp.int32, L)

    def idx_body(c, carry):
        for g in range(CHUNK // L):
            off = c * CHUNK + g * L
            p = iota + (base + off)
            fld = p % NUM_FIELDS
            idx_v[c, pl.ds(g * L, L)] = x_v[pl.ds(off, L)] + fld * VOCAB
        return carry

    lax.fori_loop(0, NCHUNK, idx_body, 0)

    def gather_body(c, carry):
        cp2 = pltpu.async_copy(t2_hbm.at[idx_v.at[c]], rows_v, sem2)
        cp1 = pltpu.async_copy(t1_hbm.at[idx_v.at[c]], e1_v, sem1)
        cp2.wait()
        cp1.wait()
        pltpu.sync_copy(rows_v, out2_hbm.at[pl.ds(base + c * CHUNK, CHUNK)])
        pltpu.sync_copy(e1_v, out1_hbm.at[pl.ds(base + c * CHUNK, CHUNK)])
        return carry

    lax.fori_loop(0, NCHUNK, gather_body, 0)


@jax.jit
def _sc_gather(x_flat, t1_flat, t2_flat):
    mesh = plsc.VectorSubcoreMesh(core_axis_name="c", subcore_axis_name="s",
                                  num_cores=NC, num_subcores=NS)
    return pl.kernel(
        _sc_gather_body,
        out_type=[
            jax.ShapeDtypeStruct((TOTAL, EMB), jnp.float32),
            jax.ShapeDtypeStruct((TOTAL,), jnp.float32),
        ],
        mesh=mesh,
        scratch_types=[
            pltpu.VMEM((PER_W,), jnp.int32),
            pltpu.VMEM((NCHUNK, CHUNK), jnp.int32),
            pltpu.VMEM((CHUNK, EMB), jnp.float32),
            pltpu.VMEM((CHUNK,), jnp.float32),
            pltpu.SemaphoreType.DMA,
            pltpu.SemaphoreType.DMA,
            pltpu.SemaphoreType.DMA,
        ],
        compiler_params=pltpu.CompilerParams(use_tc_tiling_on_sc=False),
    )(x_flat, t1_flat, t2_flat)


def _tc_dense_body(emb_ref, e1_ref, w1_ref, b1_ref, w2_ref, b2_ref,
                   w3_ref, b3_ref, out_ref):
    emb = emb_ref[...]
    r = lax.broadcasted_iota(jnp.int32, (INPUT_DIM, EMB), 0)
    d = lax.broadcasted_iota(jnp.int32, (INPUT_DIM, EMB), 1)
    s_mat = ((r % EMB) == d).astype(jnp.float32)
    sum_emb = jnp.dot(emb, s_mat, preferred_element_type=jnp.float32)
    fm2 = 0.5 * (jnp.sum(sum_emb * sum_emb, axis=1, keepdims=True)
                 - jnp.sum(emb * emb, axis=1, keepdims=True))
    fm1 = jnp.sum(e1_ref[...], axis=1, keepdims=True)
    h = jnp.dot(emb, w1_ref[...], preferred_element_type=jnp.float32)
    h = jnp.maximum(h + b1_ref[...], 0.0)
    h = jnp.dot(h, w2_ref[...], preferred_element_type=jnp.float32)
    h = jnp.maximum(h + b2_ref[...], 0.0)
    dnn = jnp.dot(h, w3_ref[...], preferred_element_type=jnp.float32)
    total = fm1 + fm2 + dnn + b3_ref[...]
    out_ref[...] = jax.nn.sigmoid(total)


def _tc_dense(emb, e1, w1f, b1f, w2f, b2f, w3f, b3f):
    bt = 2048
    grid = (BATCH // bt,)
    full = lambda shape: pl.BlockSpec(shape, lambda i: (0, 0))
    return pl.pallas_call(
        _tc_dense_body,
        grid=grid,
        in_specs=[
            pl.BlockSpec((bt, INPUT_DIM), lambda i: (i, 0)),
            pl.BlockSpec((bt, NUM_FIELDS), lambda i: (i, 0)),
            full((INPUT_DIM, 32)),
            full((1, 32)),
            full((32, 16)),
            full((1, 16)),
            full((16, 1)),
            full((1, 1)),
        ],
        out_specs=pl.BlockSpec((bt, 1), lambda i: (i, 0)),
        out_shape=jax.ShapeDtypeStruct((BATCH, 1), jnp.float32),
    )(emb, e1, w1f, b1f, w2f, b2f, w3f, b3f)


def kernel(x, first_tables, second_tables, W1, b1, g1, be1, rm1, rv1,
           W2, b2, g2, be2, rm2, rv2, W3, b3):
    eps = 1e-5
    x_flat = x.reshape(-1)
    t1_flat = first_tables.reshape(NUM_FIELDS * VOCAB)
    t2_flat = second_tables.reshape(NUM_FIELDS * VOCAB, EMB)

    emb_flat, e1_flat = _sc_gather(x_flat, t1_flat, t2_flat)
    emb = emb_flat.reshape(BATCH, INPUT_DIM)
    e1 = e1_flat.reshape(BATCH, NUM_FIELDS)

    a1 = g1 / jnp.sqrt(rv1 + eps)
    w1f = W1.T * a1[None, :]
    b1f = ((b1 - rm1) * a1 + be1)[None, :]
    a2 = g2 / jnp.sqrt(rv2 + eps)
    w2f = W2.T * a2[None, :]
    b2f = ((b2 - rm2) * a2 + be2)[None, :]
    w3f = W3.T
    b3f = b3[None, :]

    return _tc_dense(emb, e1, w1f, b1f, w2f, b2f, w3f, b3f)


# speedup vs baseline: 4.1266x; 4.1266x over previous
import jax
import jax.numpy as jnp
from jax import lax
from jax.experimental import pallas as pl
from jax.experimental.pallas import tpu as pltpu
from jax.experimental.pallas import tpu_sc as plsc

NUM_FIELDS = 26
VOCAB = 100000
EMB = 16
BATCH = 16384
INPUT_DIM = NUM_FIELDS * EMB

NC, NS, L = 2, 16, 16
NW = NC * NS
PLANES = INPUT_DIM
PPW = PLANES // NW
NGRP = BATCH // L


def _sc_gather_body(x_hbm, t1_hbm, t2_hbm, out2_hbm, out1_hbm,
                    plane_v, buf_v, sem_p, sem_x):
    wid = lax.axis_index("s") * NC + lax.axis_index("c")

    def plane_body(i, carry):
        fd = wid * PPW + i
        f = fd // EMB
        d = fd % EMB
        cp_p = pltpu.async_copy(t2_hbm.at[f, d], plane_v, sem_p)
        cp_x = pltpu.async_copy(x_hbm.at[f], buf_v, sem_x)
        cp_p.wait()
        cp_x.wait()

        def grp(j, carry):
            idx = buf_v[pl.ds(j * L, L)]
            val = plsc.load_gather(plane_v, [idx])
            buf_v[pl.ds(j * L, L)] = plsc.bitcast(val, jnp.int32)
            return carry

        lax.fori_loop(0, NGRP, grp, 0)
        pltpu.sync_copy(buf_v, out2_hbm.at[fd])
        return carry

    lax.fori_loop(0, PPW, plane_body, 0)

    @pl.when(wid < NUM_FIELDS)
    def _():
        cp_p = pltpu.async_copy(t1_hbm.at[wid], plane_v, sem_p)
        cp_x = pltpu.async_copy(x_hbm.at[wid], buf_v, sem_x)
        cp_p.wait()
        cp_x.wait()

        def grp(j, carry):
            idx = buf_v[pl.ds(j * L, L)]
            val = plsc.load_gather(plane_v, [idx])
            buf_v[pl.ds(j * L, L)] = plsc.bitcast(val, jnp.int32)
            return carry

        lax.fori_loop(0, NGRP, grp, 0)
        pltpu.sync_copy(buf_v, out1_hbm.at[wid])


@jax.jit
def _sc_gather(x_t, t1_t, t2_t):
    mesh = plsc.VectorSubcoreMesh(core_axis_name="c", subcore_axis_name="s",
                                  num_cores=NC, num_subcores=NS)
    return pl.kernel(
        _sc_gather_body,
        out_type=[
            jax.ShapeDtypeStruct((PLANES, BATCH), jnp.int32),
            jax.ShapeDtypeStruct((NUM_FIELDS, BATCH), jnp.int32),
        ],
        mesh=mesh,
        scratch_types=[
            pltpu.VMEM((VOCAB,), jnp.float32),
            pltpu.VMEM((BATCH,), jnp.int32),
            pltpu.SemaphoreType.DMA,
            pltpu.SemaphoreType.DMA,
        ],
        compiler_params=pltpu.CompilerParams(use_tc_tiling_on_sc=True,
                                             needs_layout_passes=False),
    )(x_t, t1_t, t2_t)


def _tc_dense_body(emb_ref, e1_ref, w1_ref, b1_ref, w2_ref, b2_ref,
                   w3_ref, b3_ref, out_ref):
    e = emb_ref[...]
    d = lax.broadcasted_iota(jnp.int32, (EMB, INPUT_DIM), 0)
    k = lax.broadcasted_iota(jnp.int32, (EMB, INPUT_DIM), 1)
    s_mat = ((k % EMB) == d).astype(jnp.float32)
    sum_emb = jnp.dot(s_mat, e, preferred_element_type=jnp.float32)
    fm2 = 0.5 * (jnp.sum(sum_emb * sum_emb, axis=0, keepdims=True)
                 - jnp.sum(e * e, axis=0, keepdims=True))
    fm1 = jnp.sum(e1_ref[...], axis=0, keepdims=True)
    h = jnp.dot(w1_ref[...], e, preferred_element_type=jnp.float32)
    h = jnp.maximum(h + b1_ref[...], 0.0)
    h = jnp.dot(w2_ref[...], h, preferred_element_type=jnp.float32)
    h = jnp.maximum(h + b2_ref[...], 0.0)
    dnn = jnp.dot(w3_ref[...], h, preferred_element_type=jnp.float32)
    total = fm1 + fm2 + dnn + b3_ref[...]
    out_ref[...] = jax.nn.sigmoid(total)


def _tc_dense(emb_t, e1_t, w1f, b1f, w2f, b2f, w3f, b3f):
    bt = 2048
    grid = (BATCH // bt,)
    full = lambda shape: pl.BlockSpec(shape, lambda i: (0, 0))
    return pl.pallas_call(
        _tc_dense_body,
        grid=grid,
        in_specs=[
            pl.BlockSpec((INPUT_DIM, bt), lambda i: (0, i)),
            pl.BlockSpec((NUM_FIELDS, bt), lambda i: (0, i)),
            full((32, INPUT_DIM)),
            full((32, 1)),
            full((16, 32)),
            full((16, 1)),
            full((1, 16)),
            full((1, 1)),
        ],
        out_specs=pl.BlockSpec((1, bt), lambda i: (0, i)),
        out_shape=jax.ShapeDtypeStruct((1, BATCH), jnp.float32),
    )(emb_t, e1_t, w1f, b1f, w2f, b2f, w3f, b3f)


def kernel(x, first_tables, second_tables, W1, b1, g1, be1, rm1, rv1,
           W2, b2, g2, be2, rm2, rv2, W3, b3):
    eps = 1e-5
    x_t = x.T
    t1_t = first_tables.transpose(0, 2, 1).reshape(NUM_FIELDS, VOCAB)
    t2_t = second_tables.transpose(0, 2, 1)

    emb_i, e1_i = _sc_gather(x_t, t1_t, t2_t)
    emb_t = lax.bitcast_convert_type(emb_i, jnp.float32)
    e1_t = lax.bitcast_convert_type(e1_i, jnp.float32)

    a1 = g1 / jnp.sqrt(rv1 + eps)
    w1f = W1 * a1[:, None]
    b1f = ((b1 - rm1) * a1 + be1)[:, None]
    a2 = g2 / jnp.sqrt(rv2 + eps)
    w2f = W2 * a2[:, None]
    b2f = ((b2 - rm2) * a2 + be2)[:, None]

    out = _tc_dense(emb_t, e1_t, w1f, b1f, w2f, b2f, W3, b3[:, None])
    return out.reshape(BATCH, 1)


# speedup vs baseline: 5.5748x; 1.3509x over previous
import jax
import jax.numpy as jnp
from jax import lax
from jax.experimental import pallas as pl
from jax.experimental.pallas import tpu as pltpu
from jax.experimental.pallas import tpu_sc as plsc

NUM_FIELDS = 26
VOCAB = 100000
EMB = 16
BATCH = 16384
INPUT_DIM = NUM_FIELDS * EMB

NC, NS, L = 2, 16, 16
NW = NC * NS
PLANES = INPUT_DIM
PPW = PLANES // NW
NGRP = BATCH // L
UNROLL = 8


def _sc_gather_body(x_hbm, t1_hbm, t2_hbm, out2_hbm, out1_hbm,
                    plane_v, buf_v, sem_p, sem_x):
    wid = lax.axis_index("s") * NC + lax.axis_index("c")

    def plane_body(i, carry):
        fd = wid * PPW + i
        f = fd // EMB
        d = fd % EMB
        cp_p = pltpu.async_copy(t2_hbm.at[f, d], plane_v, sem_p)
        cp_x = pltpu.async_copy(x_hbm.at[f], buf_v, sem_x)
        cp_p.wait()
        cp_x.wait()

        def grp(j, carry):
            for u in range(UNROLL):
                s = pl.ds(j * L * UNROLL + u * L, L)
                idx = buf_v[s]
                val = plsc.load_gather(plane_v, [idx])
                buf_v[s] = plsc.bitcast(val, jnp.int32)
            return carry

        lax.fori_loop(0, NGRP // UNROLL, grp, 0)
        pltpu.sync_copy(buf_v, out2_hbm.at[fd])
        return carry

    lax.fori_loop(0, PPW, plane_body, 0)

    @pl.when(wid < NUM_FIELDS)
    def _():
        cp_p = pltpu.async_copy(t1_hbm.at[wid], plane_v, sem_p)
        cp_x = pltpu.async_copy(x_hbm.at[wid], buf_v, sem_x)
        cp_p.wait()
        cp_x.wait()

        def grp(j, carry):
            for u in range(UNROLL):
                s = pl.ds(j * L * UNROLL + u * L, L)
                idx = buf_v[s]
                val = plsc.load_gather(plane_v, [idx])
                buf_v[s] = plsc.bitcast(val, jnp.int32)
            return carry

        lax.fori_loop(0, NGRP // UNROLL, grp, 0)
        pltpu.sync_copy(buf_v, out1_hbm.at[wid])


@jax.jit
def _sc_gather(x_t, t1_t, t2_t):
    mesh = plsc.VectorSubcoreMesh(core_axis_name="c", subcore_axis_name="s",
                                  num_cores=NC, num_subcores=NS)
    return pl.kernel(
        _sc_gather_body,
        out_type=[
            jax.ShapeDtypeStruct((PLANES, BATCH), jnp.int32),
            jax.ShapeDtypeStruct((NUM_FIELDS, BATCH), jnp.int32),
        ],
        mesh=mesh,
        scratch_types=[
            pltpu.VMEM((VOCAB,), jnp.float32),
            pltpu.VMEM((BATCH,), jnp.int32),
            pltpu.SemaphoreType.DMA,
            pltpu.SemaphoreType.DMA,
        ],
        compiler_params=pltpu.CompilerParams(use_tc_tiling_on_sc=True,
                                             needs_layout_passes=False),
    )(x_t, t1_t, t2_t)


def _tc_dense_body(emb_ref, e1_ref, w1_ref, b1_ref, w2_ref, b2_ref,
                   w3_ref, b3_ref, out_ref):
    e = emb_ref[...]
    d = lax.broadcasted_iota(jnp.int32, (EMB, INPUT_DIM), 0)
    k = lax.broadcasted_iota(jnp.int32, (EMB, INPUT_DIM), 1)
    s_mat = ((k % EMB) == d).astype(jnp.float32)
    sum_emb = jnp.dot(s_mat, e, preferred_element_type=jnp.float32)
    fm2 = 0.5 * (jnp.sum(sum_emb * sum_emb, axis=0, keepdims=True)
                 - jnp.sum(e * e, axis=0, keepdims=True))
    fm1 = jnp.sum(e1_ref[...], axis=0, keepdims=True)
    h = jnp.dot(w1_ref[...], e, preferred_element_type=jnp.float32)
    h = jnp.maximum(h + b1_ref[...], 0.0)
    h = jnp.dot(w2_ref[...], h, preferred_element_type=jnp.float32)
    h = jnp.maximum(h + b2_ref[...], 0.0)
    dnn = jnp.dot(w3_ref[...], h, preferred_element_type=jnp.float32)
    total = fm1 + fm2 + dnn + b3_ref[...]
    out_ref[...] = jax.nn.sigmoid(total)


def _tc_dense(emb_t, e1_t, w1f, b1f, w2f, b2f, w3f, b3f):
    bt = 2048
    grid = (BATCH // bt,)
    full = lambda shape: pl.BlockSpec(shape, lambda i: (0, 0))
    return pl.pallas_call(
        _tc_dense_body,
        grid=grid,
        in_specs=[
            pl.BlockSpec((INPUT_DIM, bt), lambda i: (0, i)),
            pl.BlockSpec((NUM_FIELDS, bt), lambda i: (0, i)),
            full((32, INPUT_DIM)),
            full((32, 1)),
            full((16, 32)),
            full((16, 1)),
            full((1, 16)),
            full((1, 1)),
        ],
        out_specs=pl.BlockSpec((1, bt), lambda i: (0, i)),
        out_shape=jax.ShapeDtypeStruct((1, BATCH), jnp.float32),
    )(emb_t, e1_t, w1f, b1f, w2f, b2f, w3f, b3f)


def kernel(x, first_tables, second_tables, W1, b1, g1, be1, rm1, rv1,
           W2, b2, g2, be2, rm2, rv2, W3, b3):
    eps = 1e-5
    x_t = x.T
    t1_t = first_tables.transpose(0, 2, 1).reshape(NUM_FIELDS, VOCAB)
    t2_t = second_tables.transpose(0, 2, 1)

    emb_i, e1_i = _sc_gather(x_t, t1_t, t2_t)
    emb_t = lax.bitcast_convert_type(emb_i, jnp.float32)
    e1_t = lax.bitcast_convert_type(e1_i, jnp.float32)

    a1 = g1 / jnp.sqrt(rv1 + eps)
    w1f = W1 * a1[:, None]
    b1f = ((b1 - rm1) * a1 + be1)[:, None]
    a2 = g2 / jnp.sqrt(rv2 + eps)
    w2f = W2 * a2[:, None]
    b2f = ((b2 - rm2) * a2 + be2)[:, None]

    out = _tc_dense(emb_t, e1_t, w1f, b1f, w2f, b2f, W3, b3[:, None])
    return out.reshape(BATCH, 1)


# speedup vs baseline: 6.0165x; 1.0792x over previous
import jax
import jax.numpy as jnp
from jax import lax
from jax.experimental import pallas as pl
from jax.experimental.pallas import tpu as pltpu
from jax.experimental.pallas import tpu_sc as plsc

NUM_FIELDS = 26
VOCAB = 100000
EMB = 16
BATCH = 16384
INPUT_DIM = NUM_FIELDS * EMB

NC, NS, L = 2, 16, 16
NW = NC * NS
PLANES = INPUT_DIM
PPW = PLANES // NW
NGRP = BATCH // L
UNROLL = 16


def _sc_gather_body(x_hbm, t1_hbm, t2_hbm, out2_hbm, out1_hbm,
                    plane_v, buf_v, sem_p, sem_x):
    wid = lax.axis_index("s") * NC + lax.axis_index("c")

    def plane_body(i, carry):
        fd = wid * PPW + i
        f = fd // EMB
        d = fd % EMB
        cp_p = pltpu.async_copy(t2_hbm.at[f, d], plane_v, sem_p)
        cp_x = pltpu.async_copy(x_hbm.at[f], buf_v, sem_x)
        cp_p.wait()
        cp_x.wait()

        def grp(j, carry):
            for u in range(UNROLL):
                s = pl.ds(j * L * UNROLL + u * L, L)
                idx = buf_v[s]
                val = plsc.load_gather(plane_v, [idx])
                buf_v[s] = plsc.bitcast(val, jnp.int32)
            return carry

        lax.fori_loop(0, NGRP // UNROLL, grp, 0)
        pltpu.sync_copy(buf_v, out2_hbm.at[fd])
        return carry

    lax.fori_loop(0, PPW, plane_body, 0)

    @pl.when(wid < NUM_FIELDS)
    def _():
        cp_p = pltpu.async_copy(t1_hbm.at[wid, 0], plane_v, sem_p)
        cp_x = pltpu.async_copy(x_hbm.at[wid], buf_v, sem_x)
        cp_p.wait()
        cp_x.wait()

        def grp(j, carry):
            for u in range(UNROLL):
                s = pl.ds(j * L * UNROLL + u * L, L)
                idx = buf_v[s]
                val = plsc.load_gather(plane_v, [idx])
                buf_v[s] = plsc.bitcast(val, jnp.int32)
            return carry

        lax.fori_loop(0, NGRP // UNROLL, grp, 0)
        pltpu.sync_copy(buf_v, out1_hbm.at[wid])


@jax.jit
def _sc_gather(x_t, t1_t, t2_t):
    mesh = plsc.VectorSubcoreMesh(core_axis_name="c", subcore_axis_name="s",
                                  num_cores=NC, num_subcores=NS)
    return pl.kernel(
        _sc_gather_body,
        out_type=[
            jax.ShapeDtypeStruct((PLANES, BATCH), jnp.int32),
            jax.ShapeDtypeStruct((NUM_FIELDS, BATCH), jnp.int32),
        ],
        mesh=mesh,
        scratch_types=[
            pltpu.VMEM((VOCAB,), jnp.float32),
            pltpu.VMEM((BATCH,), jnp.int32),
            pltpu.SemaphoreType.DMA,
            pltpu.SemaphoreType.DMA,
        ],
        compiler_params=pltpu.CompilerParams(use_tc_tiling_on_sc=True,
                                             needs_layout_passes=False),
    )(x_t, t1_t, t2_t)


def _tc_dense_body(emb_ref, e1_ref, w1_ref, b1_ref, w2_ref, b2_ref,
                   w3_ref, b3_ref, out_ref):
    e = emb_ref[...]
    d = lax.broadcasted_iota(jnp.int32, (EMB, INPUT_DIM), 0)
    k = lax.broadcasted_iota(jnp.int32, (EMB, INPUT_DIM), 1)
    s_mat = ((k % EMB) == d).astype(jnp.float32)
    sum_emb = jnp.dot(s_mat, e, preferred_element_type=jnp.float32)
    fm2 = 0.5 * (jnp.sum(sum_emb * sum_emb, axis=0, keepdims=True)
                 - jnp.sum(e * e, axis=0, keepdims=True))
    fm1 = jnp.sum(e1_ref[...], axis=0, keepdims=True)
    h = jnp.dot(w1_ref[...], e, preferred_element_type=jnp.float32)
    h = jnp.maximum(h + b1_ref[...], 0.0)
    h = jnp.dot(w2_ref[...], h, preferred_element_type=jnp.float32)
    h = jnp.maximum(h + b2_ref[...], 0.0)
    dnn = jnp.dot(w3_ref[...], h, preferred_element_type=jnp.float32)
    total = fm1 + fm2 + dnn + b3_ref[...]
    out_ref[...] = jax.nn.sigmoid(total)


def _tc_dense(emb_t, e1_t, w1f, b1f, w2f, b2f, w3f, b3f):
    bt = 2048
    grid = (BATCH // bt,)
    full = lambda shape: pl.BlockSpec(shape, lambda i: (0, 0))
    return pl.pallas_call(
        _tc_dense_body,
        grid=grid,
        in_specs=[
            pl.BlockSpec((INPUT_DIM, bt), lambda i: (0, i)),
            pl.BlockSpec((NUM_FIELDS, bt), lambda i: (0, i)),
            full((32, INPUT_DIM)),
            full((32, 1)),
            full((16, 32)),
            full((16, 1)),
            full((1, 16)),
            full((1, 1)),
        ],
        out_specs=pl.BlockSpec((1, bt), lambda i: (0, i)),
        out_shape=jax.ShapeDtypeStruct((1, BATCH), jnp.float32),
    )(emb_t, e1_t, w1f, b1f, w2f, b2f, w3f, b3f)


def kernel(x, first_tables, second_tables, W1, b1, g1, be1, rm1, rv1,
           W2, b2, g2, be2, rm2, rv2, W3, b3):
    eps = 1e-5
    x_t = x.T
    t1_t = first_tables.transpose(0, 2, 1)
    t2_t = second_tables.transpose(0, 2, 1)

    emb_i, e1_i = _sc_gather(x_t, t1_t, t2_t)
    emb_t = lax.bitcast_convert_type(emb_i, jnp.float32)
    e1_t = lax.bitcast_convert_type(e1_i, jnp.float32)

    a1 = g1 / jnp.sqrt(rv1 + eps)
    w1f = W1 * a1[:, None]
    b1f = ((b1 - rm1) * a1 + be1)[:, None]
    a2 = g2 / jnp.sqrt(rv2 + eps)
    w2f = W2 * a2[:, None]
    b2f = ((b2 - rm2) * a2 + be2)[:, None]

    out = _tc_dense(emb_t, e1_t, w1f, b1f, w2f, b2f, W3, b3[:, None])
    return out.reshape(BATCH, 1)


# speedup vs baseline: 6.0739x; 1.0095x over previous
import jax
import jax.numpy as jnp
from jax import lax
from jax.experimental import pallas as pl
from jax.experimental.pallas import tpu as pltpu
from jax.experimental.pallas import tpu_sc as plsc

NUM_FIELDS = 26
VOCAB = 100000
EMB = 16
BATCH = 16384
INPUT_DIM = NUM_FIELDS * EMB

NC, NS, L = 2, 16, 16
NW = NC * NS
PLANES = INPUT_DIM
PPW = PLANES // NW
NGRP = BATCH // L
UNROLL = 16


def _sc_gather_body(x_hbm, t1_hbm, t2_hbm, out2_hbm, out1_hbm,
                    plane_v, buf_v, sem_p, sem_x):
    wid = lax.axis_index("s") * NC + lax.axis_index("c")

    def plane_body(i, carry):
        fd = wid * PPW + i
        f = fd // EMB
        d = fd % EMB
        cp_p = pltpu.async_copy(t2_hbm.at[f, d], plane_v, sem_p)
        cp_x = pltpu.async_copy(x_hbm.at[f], buf_v, sem_x)
        cp_p.wait()
        cp_x.wait()

        def grp(j, carry):
            for u in range(UNROLL):
                s = pl.ds(j * L * UNROLL + u * L, L)
                idx = buf_v[s]
                val = plsc.load_gather(plane_v, [idx])
                buf_v[s] = plsc.bitcast(val, jnp.int32)
            return carry

        lax.fori_loop(0, NGRP // UNROLL, grp, 0)
        pltpu.sync_copy(buf_v, out2_hbm.at[fd])
        return carry

    lax.fori_loop(0, PPW, plane_body, 0)

    @pl.when(wid < NUM_FIELDS)
    def _():
        cp_p = pltpu.async_copy(t1_hbm.at[wid, 0], plane_v, sem_p)
        cp_x = pltpu.async_copy(x_hbm.at[wid], buf_v, sem_x)
        cp_p.wait()
        cp_x.wait()

        def grp(j, carry):
            for u in range(UNROLL):
                s = pl.ds(j * L * UNROLL + u * L, L)
                idx = buf_v[s]
                val = plsc.load_gather(plane_v, [idx])
                buf_v[s] = plsc.bitcast(val, jnp.int32)
            return carry

        lax.fori_loop(0, NGRP // UNROLL, grp, 0)
        pltpu.sync_copy(buf_v, out1_hbm.at[wid])


@jax.jit
def _sc_gather(x_t, t1_t, t2_t):
    mesh = plsc.VectorSubcoreMesh(core_axis_name="c", subcore_axis_name="s",
                                  num_cores=NC, num_subcores=NS)
    return pl.kernel(
        _sc_gather_body,
        out_type=[
            jax.ShapeDtypeStruct((PLANES, BATCH), jnp.int32),
            jax.ShapeDtypeStruct((NUM_FIELDS, BATCH), jnp.int32),
        ],
        mesh=mesh,
        scratch_types=[
            pltpu.VMEM((VOCAB,), jnp.float32),
            pltpu.VMEM((BATCH,), jnp.int32),
            pltpu.SemaphoreType.DMA,
            pltpu.SemaphoreType.DMA,
        ],
        compiler_params=pltpu.CompilerParams(use_tc_tiling_on_sc=True,
                                             needs_layout_passes=False),
    )(x_t, t1_t, t2_t)


def _tc_dense_body(emb_ref, e1_ref, w1_ref, b1_ref, w2_ref, b2_ref,
                   w3_ref, b3_ref, out_ref):
    e = emb_ref[...]
    d = lax.broadcasted_iota(jnp.int32, (EMB, INPUT_DIM), 0)
    k = lax.broadcasted_iota(jnp.int32, (EMB, INPUT_DIM), 1)
    s_mat = ((k % EMB) == d).astype(jnp.float32)
    sum_emb = jnp.dot(s_mat, e, preferred_element_type=jnp.float32)
    fm2 = 0.5 * (jnp.sum(sum_emb * sum_emb, axis=0, keepdims=True)
                 - jnp.sum(e * e, axis=0, keepdims=True))
    fm1 = jnp.sum(e1_ref[...], axis=0, keepdims=True)
    h = jnp.dot(w1_ref[...], e, preferred_element_type=jnp.float32)
    h = jnp.maximum(h + b1_ref[...], 0.0)
    h = jnp.dot(w2_ref[...], h, preferred_element_type=jnp.float32)
    h = jnp.maximum(h + b2_ref[...], 0.0)
    dnn = jnp.dot(w3_ref[...], h, preferred_element_type=jnp.float32)
    total = fm1 + fm2 + dnn + b3_ref[...]
    out_ref[...] = jax.nn.sigmoid(total)


def _tc_dense(emb_t, e1_t, w1f, b1f, w2f, b2f, w3f, b3f):
    bt = 4096
    grid = (BATCH // bt,)
    full = lambda shape: pl.BlockSpec(shape, lambda i: (0, 0))
    return pl.pallas_call(
        _tc_dense_body,
        grid=grid,
        in_specs=[
            pl.BlockSpec((INPUT_DIM, bt), lambda i: (0, i)),
            pl.BlockSpec((NUM_FIELDS, bt), lambda i: (0, i)),
            full((32, INPUT_DIM)),
            full((32, 1)),
            full((16, 32)),
            full((16, 1)),
            full((1, 16)),
            full((1, 1)),
        ],
        out_specs=pl.BlockSpec((1, bt), lambda i: (0, i)),
        out_shape=jax.ShapeDtypeStruct((1, BATCH), jnp.float32),
    )(emb_t, e1_t, w1f, b1f, w2f, b2f, w3f, b3f)


def kernel(x, first_tables, second_tables, W1, b1, g1, be1, rm1, rv1,
           W2, b2, g2, be2, rm2, rv2, W3, b3):
    eps = 1e-5
    x_t = x.T
    t1_t = first_tables.transpose(0, 2, 1)
    t2_t = second_tables.transpose(0, 2, 1)

    emb_i, e1_i = _sc_gather(x_t, t1_t, t2_t)
    emb_t = lax.bitcast_convert_type(emb_i, jnp.float32)
    e1_t = lax.bitcast_convert_type(e1_i, jnp.float32)

    a1 = g1 / jnp.sqrt(rv1 + eps)
    w1f = W1 * a1[:, None]
    b1f = ((b1 - rm1) * a1 + be1)[:, None]
    a2 = g2 / jnp.sqrt(rv2 + eps)
    w2f = W2 * a2[:, None]
    b2f = ((b2 - rm2) * a2 + be2)[:, None]

    out = _tc_dense(emb_t, e1_t, w1f, b1f, w2f, b2f, W3, b3[:, None])
    return out.reshape(BATCH, 1)


# speedup vs baseline: 6.5816x; 1.0836x over previous
import jax
import jax.numpy as jnp
from jax import lax
from jax.experimental import pallas as pl
from jax.experimental.pallas import tpu as pltpu
from jax.experimental.pallas import tpu_sc as plsc

NUM_FIELDS = 26
VOCAB = 100000
EMB = 16
BATCH = 16384
INPUT_DIM = NUM_FIELDS * EMB

NC, NS, L = 2, 16, 16
NW = NC * NS
PLANES = INPUT_DIM
PPW = PLANES // NW
NGRP = BATCH // L
UNROLL = 16
CH = 4096
NCH = BATCH // CH
CHG = CH // L


def _sc_gather_body(x_hbm, t1_hbm, t2_hbm, out2_hbm, out1_hbm,
                    plane_v, idx_v, val0_v, val1_v, sem_p, sem_x, sem_o0, sem_o1):
    wid = lax.axis_index("s") * NC + lax.axis_index("c")
    osems = (sem_o0, sem_o1)
    vbufs = (val0_v, val1_v)

    def gather_plane(out_row):
        for c in range(NCH):
            vb = vbufs[c % 2]

            def grp(j, carry):
                for u in range(UNROLL):
                    g = j * UNROLL + u
                    idx = idx_v[pl.ds((c * CHG + g) * L, L)]
                    val = plsc.load_gather(plane_v, [idx])
                    vb[pl.ds(g * L, L)] = plsc.bitcast(val, jnp.int32)
                return carry

            dst = out_row.at[pl.ds(c * CH, CH)]
            if c >= 2:
                pltpu.make_async_copy(vb, out_row.at[pl.ds((c - 2) * CH, CH)],
                                      osems[c % 2]).wait()
            lax.fori_loop(0, CHG // UNROLL, grp, 0)
            pltpu.async_copy(vb, dst, osems[c % 2])
        for c in range(NCH - 2, NCH):
            pltpu.make_async_copy(vbufs[c % 2],
                                  out_row.at[pl.ds(c * CH, CH)],
                                  osems[c % 2]).wait()

    def stage_x(f):
        pltpu.async_copy(x_hbm.at[f], idx_v, sem_x).wait()

    f0 = (wid * PPW) // EMB
    pltpu.async_copy(t2_hbm.at[f0, (wid * PPW) % EMB], plane_v, sem_p)
    stage_x(f0)

    def plane_body(i, carry):
        fd = wid * PPW + i
        f = fd // EMB
        d = fd % EMB

        @pl.when((i > 0) & (d == 0))
        def _():
            stage_x(f)

        pltpu.make_async_copy(t2_hbm.at[f, d], plane_v, sem_p).wait()
        gather_plane(out2_hbm.at[fd])

        nfd = fd + 1

        @pl.when(i + 1 < PPW)
        def _():
            pltpu.async_copy(t2_hbm.at[nfd // EMB, nfd % EMB], plane_v, sem_p)

        @pl.when((i + 1 == PPW) & (wid < NUM_FIELDS))
        def _():
            pltpu.async_copy(t1_hbm.at[wid, 0], plane_v, sem_p)

        return carry

    lax.fori_loop(0, PPW, plane_body, 0)

    @pl.when(wid < NUM_FIELDS)
    def _():
        stage_x(wid)
        pltpu.make_async_copy(t1_hbm.at[wid, 0], plane_v, sem_p).wait()
        gather_plane(out1_hbm.at[wid])


@jax.jit
def _sc_gather(x_t, t1_t, t2_t):
    mesh = plsc.VectorSubcoreMesh(core_axis_name="c", subcore_axis_name="s",
                                  num_cores=NC, num_subcores=NS)
    return pl.kernel(
        _sc_gather_body,
        out_type=[
            jax.ShapeDtypeStruct((PLANES, BATCH), jnp.int32),
            jax.ShapeDtypeStruct((NUM_FIELDS, BATCH), jnp.int32),
        ],
        mesh=mesh,
        scratch_types=[
            pltpu.VMEM((VOCAB,), jnp.float32),
            pltpu.VMEM((BATCH,), jnp.int32),
            pltpu.VMEM((CH,), jnp.int32),
            pltpu.VMEM((CH,), jnp.int32),
            pltpu.SemaphoreType.DMA,
            pltpu.SemaphoreType.DMA,
            pltpu.SemaphoreType.DMA,
            pltpu.SemaphoreType.DMA,
        ],
        compiler_params=pltpu.CompilerParams(use_tc_tiling_on_sc=True,
                                             needs_layout_passes=False),
    )(x_t, t1_t, t2_t)


def _tc_dense_body(emb_ref, e1_ref, w1_ref, b1_ref, w2_ref, b2_ref,
                   w3_ref, b3_ref, out_ref):
    e = emb_ref[...]
    d = lax.broadcasted_iota(jnp.int32, (EMB, INPUT_DIM), 0)
    k = lax.broadcasted_iota(jnp.int32, (EMB, INPUT_DIM), 1)
    s_mat = ((k % EMB) == d).astype(jnp.float32)
    sum_emb = jnp.dot(s_mat, e, preferred_element_type=jnp.float32)
    fm2 = 0.5 * (jnp.sum(sum_emb * sum_emb, axis=0, keepdims=True)
                 - jnp.sum(e * e, axis=0, keepdims=True))
    fm1 = jnp.sum(e1_ref[...], axis=0, keepdims=True)
    h = jnp.dot(w1_ref[...], e, preferred_element_type=jnp.float32)
    h = jnp.maximum(h + b1_ref[...], 0.0)
    h = jnp.dot(w2_ref[...], h, preferred_element_type=jnp.float32)
    h = jnp.maximum(h + b2_ref[...], 0.0)
    dnn = jnp.dot(w3_ref[...], h, preferred_element_type=jnp.float32)
    total = fm1 + fm2 + dnn + b3_ref[...]
    out_ref[...] = jax.nn.sigmoid(total)


def _tc_dense(emb_t, e1_t, w1f, b1f, w2f, b2f, w3f, b3f):
    bt = 4096
    grid = (BATCH // bt,)
    full = lambda shape: pl.BlockSpec(shape, lambda i: (0, 0))
    return pl.pallas_call(
        _tc_dense_body,
        grid=grid,
        in_specs=[
            pl.BlockSpec((INPUT_DIM, bt), lambda i: (0, i)),
            pl.BlockSpec((NUM_FIELDS, bt), lambda i: (0, i)),
            full((32, INPUT_DIM)),
            full((32, 1)),
            full((16, 32)),
            full((16, 1)),
            full((1, 16)),
            full((1, 1)),
        ],
        out_specs=pl.BlockSpec((1, bt), lambda i: (0, i)),
        out_shape=jax.ShapeDtypeStruct((1, BATCH), jnp.float32),
    )(emb_t, e1_t, w1f, b1f, w2f, b2f, w3f, b3f)


def kernel(x, first_tables, second_tables, W1, b1, g1, be1, rm1, rv1,
           W2, b2, g2, be2, rm2, rv2, W3, b3):
    eps = 1e-5
    x_t = x.T
    t1_t = first_tables.transpose(0, 2, 1)
    t2_t = second_tables.transpose(0, 2, 1)

    emb_i, e1_i = _sc_gather(x_t, t1_t, t2_t)
    emb_t = lax.bitcast_convert_type(emb_i, jnp.float32)
    e1_t = lax.bitcast_convert_type(e1_i, jnp.float32)

    a1 = g1 / jnp.sqrt(rv1 + eps)
    w1f = W1 * a1[:, None]
    b1f = ((b1 - rm1) * a1 + be1)[:, None]
    a2 = g2 / jnp.sqrt(rv2 + eps)
    w2f = W2 * a2[:, None]
    b2f = ((b2 - rm2) * a2 + be2)[:, None]

    out = _tc_dense(emb_t, e1_t, w1f, b1f, w2f, b2f, W3, b3[:, None])
    return out.reshape(BATCH, 1)


# speedup vs baseline: 8.4067x; 1.2773x over previous
import jax
import jax.numpy as jnp
from jax import lax
from jax.experimental import pallas as pl
from jax.experimental.pallas import tpu as pltpu
from jax.experimental.pallas import tpu_sc as plsc

NUM_FIELDS = 26
VOCAB = 100000
EMB = 16
BATCH = 16384
INPUT_DIM = NUM_FIELDS * EMB

NC, NS, L = 2, 16, 16
NW = NC * NS
PLANES = INPUT_DIM
PPW = PLANES // NW
NGRP = BATCH // L
UNROLL = 16
CH = 4096
NCH = BATCH // CH
CHG = CH // L


def _sc_gather_body(x_hbm, t1_hbm, t2_hbm, out2_hbm, out1_hbm,
                    plane_v, idx_v, val0_v, val1_v, sem_p, sem_x, sem_o0, sem_o1):
    wid = lax.axis_index("s") * NC + lax.axis_index("c")
    osems = (sem_o0, sem_o1)
    vbufs = (val0_v, val1_v)

    def gather_plane(out_row):
        for c in range(NCH):
            vb = vbufs[c % 2]

            dst = out_row.at[pl.ds(c * CH, CH)]
            if c >= 2:
                pltpu.make_async_copy(vb, out_row.at[pl.ds((c - 2) * CH, CH)],
                                      osems[c % 2]).wait()

            @plsc.parallel_loop(0, CHG, unroll=UNROLL)
            def _(g):
                idx = idx_v[pl.ds((c * CHG + g) * L, L)]
                val = plsc.load_gather(plane_v, [idx])
                vb[pl.ds(g * L, L)] = plsc.bitcast(val, jnp.int32)

            pltpu.async_copy(vb, dst, osems[c % 2])
        for c in range(NCH - 2, NCH):
            pltpu.make_async_copy(vbufs[c % 2],
                                  out_row.at[pl.ds(c * CH, CH)],
                                  osems[c % 2]).wait()

    def stage_x(f):
        pltpu.async_copy(x_hbm.at[f], idx_v, sem_x).wait()

    f0 = (wid * PPW) // EMB
    pltpu.async_copy(t2_hbm.at[f0, (wid * PPW) % EMB], plane_v, sem_p)
    stage_x(f0)

    def plane_body(i, carry):
        fd = wid * PPW + i
        f = fd // EMB
        d = fd % EMB

        @pl.when((i > 0) & (d == 0))
        def _():
            stage_x(f)

        pltpu.make_async_copy(t2_hbm.at[f, d], plane_v, sem_p).wait()
        gather_plane(out2_hbm.at[fd])

        nfd = fd + 1

        @pl.when(i + 1 < PPW)
        def _():
            pltpu.async_copy(t2_hbm.at[nfd // EMB, nfd % EMB], plane_v, sem_p)

        @pl.when((i + 1 == PPW) & (wid < NUM_FIELDS))
        def _():
            pltpu.async_copy(t1_hbm.at[wid, 0], plane_v, sem_p)

        return carry

    lax.fori_loop(0, PPW, plane_body, 0)

    @pl.when(wid < NUM_FIELDS)
    def _():
        stage_x(wid)
        pltpu.make_async_copy(t1_hbm.at[wid, 0], plane_v, sem_p).wait()
        gather_plane(out1_hbm.at[wid])


@jax.jit
def _sc_gather(x_t, t1_t, t2_t):
    mesh = plsc.VectorSubcoreMesh(core_axis_name="c", subcore_axis_name="s",
                                  num_cores=NC, num_subcores=NS)
    return pl.kernel(
        _sc_gather_body,
        out_type=[
            jax.ShapeDtypeStruct((PLANES, BATCH), jnp.int32),
            jax.ShapeDtypeStruct((NUM_FIELDS, BATCH), jnp.int32),
        ],
        mesh=mesh,
        scratch_types=[
            pltpu.VMEM((VOCAB,), jnp.float32),
            pltpu.VMEM((BATCH,), jnp.int32),
            pltpu.VMEM((CH,), jnp.int32),
            pltpu.VMEM((CH,), jnp.int32),
            pltpu.SemaphoreType.DMA,
            pltpu.SemaphoreType.DMA,
            pltpu.SemaphoreType.DMA,
            pltpu.SemaphoreType.DMA,
        ],
        compiler_params=pltpu.CompilerParams(use_tc_tiling_on_sc=True,
                                             needs_layout_passes=False),
    )(x_t, t1_t, t2_t)


def _tc_dense_body(emb_ref, e1_ref, w1_ref, b1_ref, w2_ref, b2_ref,
                   w3_ref, b3_ref, out_ref):
    e = emb_ref[...]
    d = lax.broadcasted_iota(jnp.int32, (EMB, INPUT_DIM), 0)
    k = lax.broadcasted_iota(jnp.int32, (EMB, INPUT_DIM), 1)
    s_mat = ((k % EMB) == d).astype(jnp.float32)
    sum_emb = jnp.dot(s_mat, e, preferred_element_type=jnp.float32)
    fm2 = 0.5 * (jnp.sum(sum_emb * sum_emb, axis=0, keepdims=True)
                 - jnp.sum(e * e, axis=0, keepdims=True))
    fm1 = jnp.sum(e1_ref[...], axis=0, keepdims=True)
    h = jnp.dot(w1_ref[...], e, preferred_element_type=jnp.float32)
    h = jnp.maximum(h + b1_ref[...], 0.0)
    h = jnp.dot(w2_ref[...], h, preferred_element_type=jnp.float32)
    h = jnp.maximum(h + b2_ref[...], 0.0)
    dnn = jnp.dot(w3_ref[...], h, preferred_element_type=jnp.float32)
    total = fm1 + fm2 + dnn + b3_ref[...]
    out_ref[...] = jax.nn.sigmoid(total)


def _tc_dense(emb_t, e1_t, w1f, b1f, w2f, b2f, w3f, b3f):
    bt = 4096
    grid = (BATCH // bt,)
    full = lambda shape: pl.BlockSpec(shape, lambda i: (0, 0))
    return pl.pallas_call(
        _tc_dense_body,
        grid=grid,
        in_specs=[
            pl.BlockSpec((INPUT_DIM, bt), lambda i: (0, i)),
            pl.BlockSpec((NUM_FIELDS, bt), lambda i: (0, i)),
            full((32, INPUT_DIM)),
            full((32, 1)),
            full((16, 32)),
            full((16, 1)),
            full((1, 16)),
            full((1, 1)),
        ],
        out_specs=pl.BlockSpec((1, bt), lambda i: (0, i)),
        out_shape=jax.ShapeDtypeStruct((1, BATCH), jnp.float32),
    )(emb_t, e1_t, w1f, b1f, w2f, b2f, w3f, b3f)


def kernel(x, first_tables, second_tables, W1, b1, g1, be1, rm1, rv1,
           W2, b2, g2, be2, rm2, rv2, W3, b3):
    eps = 1e-5
    x_t = x.T
    t1_t = first_tables.transpose(0, 2, 1)
    t2_t = second_tables.transpose(0, 2, 1)

    emb_i, e1_i = _sc_gather(x_t, t1_t, t2_t)
    emb_t = lax.bitcast_convert_type(emb_i, jnp.float32)
    e1_t = lax.bitcast_convert_type(e1_i, jnp.float32)

    a1 = g1 / jnp.sqrt(rv1 + eps)
    w1f = W1 * a1[:, None]
    b1f = ((b1 - rm1) * a1 + be1)[:, None]
    a2 = g2 / jnp.sqrt(rv2 + eps)
    w2f = W2 * a2[:, None]
    b2f = ((b2 - rm2) * a2 + be2)[:, None]

    out = _tc_dense(emb_t, e1_t, w1f, b1f, w2f, b2f, W3, b3[:, None])
    return out.reshape(BATCH, 1)
